# baseline (speedup 1.0000x reference)
import functools

import jax
import jax.numpy as jnp
from jax import lax
from jax.experimental import pallas as pl
from jax.experimental.pallas import tpu as pltpu
from jax.experimental.pallas import tpu_sc as plsc

_B, _S, _D = 4096, 50, 128
_SP = 56
_NW = 32
_BPW = _B // _NW
_IDX_W = _BPW * _SP
_NBB = 8
_CROWS = _NBB * _SP
_GCH = _BPW // _NBB

_mesh = plsc.VectorSubcoreMesh(core_axis_name="c", subcore_axis_name="s")


@functools.partial(
    pl.kernel,
    mesh=_mesh,
    out_type=jax.ShapeDtypeStruct((_B * _SP, _D), jnp.float32),
    scratch_types=[
        pltpu.VMEM((_IDX_W,), jnp.int32),
        pltpu.VMEM((2, _CROWS, _D), jnp.float32),
        pltpu.SemaphoreType.DMA,
        pltpu.SemaphoreType.DMA,
    ],
    compiler_params=pltpu.CompilerParams(use_tc_tiling_on_sc=True),
)
def _gather_kernel(ids_hbm, table_hbm, out_hbm, idx_v, rows_v, gsem, ssem):
    wid = lax.axis_index("s") * 2 + lax.axis_index("c")
    base_r = wid * _IDX_W
    pltpu.sync_copy(ids_hbm.at[pl.ds(base_r, _IDX_W)], idx_v)

    def gather(c, buf):
        pltpu.async_copy(
            table_hbm.at[idx_v.at[pl.ds(c * _CROWS, _CROWS)]],
            rows_v.at[buf], gsem)

    def gwait(buf):
        pltpu.make_async_copy(
            table_hbm.at[pl.ds(0, _CROWS)], rows_v.at[buf], gsem).wait()

    def scatter(c, buf):
        pltpu.async_copy(
            rows_v.at[buf],
            out_hbm.at[pl.ds(base_r + c * _CROWS, _CROWS)], ssem)

    def swait(buf):
        pltpu.make_async_copy(
            rows_v.at[buf], out_hbm.at[pl.ds(base_r, _CROWS)], ssem).wait()

    gather(0, 0)
    gather(1, 1)

    def body(i, carry):
        c = 2 * i
        gwait(0)
        scatter(c, 0)
        swait(0)
        gather(c + 2, 0)
        gwait(1)
        scatter(c + 1, 1)
        swait(1)
        gather(c + 3, 1)
        return carry

    lax.fori_loop(0, (_GCH - 2) // 2, body, 0)

    gwait(0)
    scatter(_GCH - 2, 0)
    gwait(1)
    scatter(_GCH - 1, 1)
    swait(0)
    swait(1)


def kernel(word_ids, table):
    ids = word_ids.astype(jnp.int32)
    ids_pad = jnp.pad(ids, ((0, 0), (0, _SP - _S))).reshape(-1)
    out = _gather_kernel(ids_pad, table)
    return out.reshape(_B, _SP, _D)[:, :_S, :]


# speedup vs baseline: 6.4216x; 6.4216x over previous
import functools

import jax
import jax.numpy as jnp
from jax import lax
from jax.experimental import pallas as pl
from jax.experimental.pallas import tpu as pltpu
from jax.experimental.pallas import tpu_sc as plsc

_B, _S, _D = 4096, 50, 128
_SP = 56
_NW = 32
_BPW = _B // _NW
_IDX_W = _BPW * _SP
_NBB = 8
_CROWS = _NBB * _SP
_GCH = _BPW // _NBB

_mesh = plsc.VectorSubcoreMesh(core_axis_name="c", subcore_axis_name="s")


@functools.partial(
    pl.kernel,
    mesh=_mesh,
    out_type=jax.ShapeDtypeStruct((_B * _SP, _D), jnp.float32),
    scratch_types=[
        pltpu.VMEM((_IDX_W,), jnp.int32),
        pltpu.VMEM((2, _CROWS, _D), jnp.float32),
        pltpu.SemaphoreType.DMA,
        pltpu.SemaphoreType.DMA,
    ],
    compiler_params=pltpu.CompilerParams(use_tc_tiling_on_sc=True),
)
def _gather_kernel(ids_hbm, table_hbm, out_hbm, idx_v, rows_v, gsem, ssem):
    wid = lax.axis_index("s") * 2 + lax.axis_index("c")
    base_r = wid * _IDX_W
    pltpu.sync_copy(ids_hbm.at[pl.ds(base_r, _IDX_W)], idx_v)

    def gather(c, buf):
        pltpu.async_copy(
            table_hbm.at[idx_v.at[pl.ds(c * _CROWS, _CROWS)]],
            rows_v.at[buf], gsem)

    def gwait(buf):
        pltpu.make_async_copy(
            table_hbm.at[pl.ds(0, _CROWS)], rows_v.at[buf], gsem).wait()

    def scatter(c, buf):
        pltpu.async_copy(
            rows_v.at[buf],
            out_hbm.at[pl.ds(base_r + c * _CROWS, _CROWS)], ssem)

    def swait(buf):
        pltpu.make_async_copy(
            rows_v.at[buf], out_hbm.at[pl.ds(base_r, _CROWS)], ssem).wait()

    gather(0, 0)
    gather(1, 1)

    def body(i, carry):
        c = 2 * i
        gwait(0)
        scatter(c, 0)
        swait(0)
        gather(c + 2, 0)
        gwait(1)
        scatter(c + 1, 1)
        swait(1)
        gather(c + 3, 1)
        return carry

    lax.fori_loop(0, (_GCH - 2) // 2, body, 0)

    gwait(0)
    scatter(_GCH - 2, 0)
    gwait(1)
    scatter(_GCH - 1, 1)
    swait(0)
    swait(1)


def kernel(word_ids, table):
    ids = word_ids.astype(jnp.int32)
    ids_pad = jnp.concatenate([ids, ids[:, : _SP - _S]], axis=1).reshape(-1)
    out = _gather_kernel(ids_pad, table)
    return out.reshape(_B, _SP, _D)[:, :_S, :]


# speedup vs baseline: 7.4481x; 1.1599x over previous
import functools

import jax
import jax.numpy as jnp
from jax import lax
from jax.experimental import pallas as pl
from jax.experimental.pallas import tpu as pltpu
from jax.experimental.pallas import tpu_sc as plsc

_B, _S, _D = 4096, 50, 128
_SP = 56
_NW = 32
_BPW = _B // _NW
_IDX_W = _BPW * _SP
_NBB = 8
_CROWS = _NBB * _SP
_GCH = _BPW // _NBB

_mesh = plsc.VectorSubcoreMesh(core_axis_name="c", subcore_axis_name="s")


@functools.partial(
    pl.kernel,
    mesh=_mesh,
    out_type=jax.ShapeDtypeStruct((_B, _S, _D), jnp.float32),
    scratch_types=[
        pltpu.VMEM((_IDX_W,), jnp.int32),
        pltpu.VMEM((2, _CROWS, _D), jnp.float32),
        pltpu.SemaphoreType.DMA,
        pltpu.SemaphoreType.DMA,
    ],
    compiler_params=pltpu.CompilerParams(use_tc_tiling_on_sc=True),
)
def _gather_kernel(ids_hbm, table_hbm, out_hbm, idx_v, rows_v, gsem, ssem):
    wid = lax.axis_index("s") * 2 + lax.axis_index("c")
    base_r = wid * _IDX_W
    pltpu.sync_copy(ids_hbm.at[pl.ds(base_r, _IDX_W)], idx_v)

    def gather(c, buf):
        pltpu.async_copy(
            table_hbm.at[idx_v.at[pl.ds(c * _CROWS, _CROWS)]],
            rows_v.at[buf], gsem)

    def gwait(buf):
        pltpu.make_async_copy(
            table_hbm.at[pl.ds(0, _CROWS)], rows_v.at[buf], gsem).wait()

    base_b = wid * _BPW

    def scatter(c, buf):
        for j in range(_NBB):
            pltpu.async_copy(
                rows_v.at[buf, pl.ds(j * _SP, _S)],
                out_hbm.at[base_b + c * _NBB + j], ssem)

    def swait(buf):
        for _ in range(_NBB):
            pltpu.make_async_copy(
                rows_v.at[buf, pl.ds(0, _S)], out_hbm.at[base_b], ssem).wait()

    gather(0, 0)
    gather(1, 1)

    def body(i, carry):
        c = 2 * i
        gwait(0)
        scatter(c, 0)
        swait(0)
        gather(c + 2, 0)
        gwait(1)
        scatter(c + 1, 1)
        swait(1)
        gather(c + 3, 1)
        return carry

    lax.fori_loop(0, (_GCH - 2) // 2, body, 0)

    gwait(0)
    scatter(_GCH - 2, 0)
    gwait(1)
    scatter(_GCH - 1, 1)
    swait(0)
    swait(1)


def kernel(word_ids, table):
    ids = word_ids.astype(jnp.int32)
    ids_pad = jnp.concatenate([ids, ids[:, : _SP - _S]], axis=1).reshape(-1)
    return _gather_kernel(ids_pad, table)


# speedup vs baseline: 13.9035x; 1.8667x over previous
import functools

import jax
import jax.numpy as jnp
from jax import lax
from jax.experimental import pallas as pl
from jax.experimental.pallas import tpu as pltpu
from jax.experimental.pallas import tpu_sc as plsc

_B, _S, _D = 4096, 50, 128
_N = _B * _S
_NW = 32
_PER_W = _N // _NW
_CHUNK = 320
_NCH = _PER_W // _CHUNK

_mesh = plsc.VectorSubcoreMesh(core_axis_name="c", subcore_axis_name="s")


@functools.partial(
    pl.kernel,
    mesh=_mesh,
    out_type=jax.ShapeDtypeStruct((_N, _D), jnp.float32),
    scratch_types=[
        pltpu.VMEM((_PER_W,), jnp.int32),
        pltpu.VMEM((3, _CHUNK, _D), jnp.float32),
        pltpu.SemaphoreType.DMA,
        pltpu.SemaphoreType.DMA,
    ],
    compiler_params=pltpu.CompilerParams(use_tc_tiling_on_sc=True),
)
def _gather_kernel(ids_hbm, table_hbm, out_hbm, idx_v, rows_v, gsem, ssem):
    wid = lax.axis_index("s") * 2 + lax.axis_index("c")
    base = wid * _PER_W
    pltpu.sync_copy(ids_hbm.at[pl.ds(base, _PER_W)], idx_v)

    def gather(c, buf):
        pltpu.async_copy(
            table_hbm.at[idx_v.at[pl.ds(c * _CHUNK, _CHUNK)]],
            rows_v.at[buf], gsem)

    def gwait(buf):
        pltpu.make_async_copy(
            table_hbm.at[pl.ds(0, _CHUNK)], rows_v.at[buf], gsem).wait()

    def scatter(c, buf):
        pltpu.async_copy(
            rows_v.at[buf], out_hbm.at[pl.ds(base + c * _CHUNK, _CHUNK)], ssem)

    def swait(buf):
        pltpu.make_async_copy(
            rows_v.at[buf], out_hbm.at[pl.ds(base, _CHUNK)], ssem).wait()

    gather(0, 0)
    gather(1, 1)

    gwait(0)
    scatter(0, 0)
    gather(2, 2)

    gwait(1)
    scatter(1, 1)
    swait(1)
    gather(3, 0)

    gwait(2)
    scatter(2, 2)
    swait(2)
    gather(4, 1)

    def body(i, carry):
        c = 3 * i
        for j in range(3):
            buf = j
            gwait(buf)
            scatter(c + j, buf)
            swait(buf)
            gather(c + j + 2, (j + 2) % 3)
        return carry

    lax.fori_loop(1, (_NCH - 2) // 3, body, 0)

    gwait(0)
    scatter(_NCH - 2, 0)
    swait(0)
    gwait(1)
    scatter(_NCH - 1, 1)
    swait(1)
    swait(2)


def kernel(word_ids, table):
    ids_t = word_ids.T.reshape(-1).astype(jnp.int32)
    out = _gather_kernel(ids_t, table)
    return jnp.transpose(out.reshape(_S, _B, _D), (1, 0, 2))
